# baseline (speedup 1.0000x reference)
import functools

import jax
import jax.numpy as jnp
from jax import lax
from jax.experimental import pallas as pl
from jax.experimental.pallas import tpu as pltpu
from jax.experimental.pallas import tpu_sc as plsc

V = 1000000
D = 16
B = 16384
L = 200

NC = 2
NS = 16
LANES = 16
NW = NC * NS
ROWS_PER_W = B // NW
CHUNK_ROWS = 128
N_CHUNKS = ROWS_PER_W // CHUNK_ROWS
CHUNK_IDX = CHUNK_ROWS * L

VP = 1048576
TC_BLK = 8192
TC_LAST = (V - 1) // TC_BLK

UNROLL = 8
assert L % UNROLL == 0


def _table_dot_body(w_ref, b_ref, emb_ref, out_ref):
    w = w_ref[0, :]
    out_ref[...] = jnp.sum(emb_ref[...] * w[None, :], axis=1) + b_ref[0]


def _fold_table(emb_table, W, b):
    wt = W.reshape(1, D)
    grid = VP // TC_BLK
    return pl.pallas_call(
        _table_dot_body,
        grid=(grid,),
        in_specs=[
            pl.BlockSpec((1, D), lambda i: (0, 0)),
            pl.BlockSpec(memory_space=pltpu.SMEM),
            pl.BlockSpec((TC_BLK, D), lambda i: (jnp.minimum(i, TC_LAST), 0)),
        ],
        out_specs=pl.BlockSpec((TC_BLK,), lambda i: (i,)),
        out_shape=jax.ShapeDtypeStruct((VP,), jnp.float32),
    )(wt, b, emb_table)


def _sc_body(t_hbm, xt_hbm, out_hbm,
             idx_a, idx_b, vals_a, vals_b, outs_v, sem_a, sem_b):
    wid = lax.axis_index("s") * NC + lax.axis_index("c")
    row0 = wid * ROWS_PER_W

    def fetch(c, idx_v, vals_v, sem):
        idx0 = (row0 + c * CHUNK_ROWS) * L
        pltpu.sync_copy(xt_hbm.at[pl.ds(idx0, CHUNK_IDX)], idx_v)
        return pltpu.async_copy(t_hbm.at[idx_v], vals_v, sem)

    def compute(c, vals_v):
        def group_body(g, _):
            base = g * (LANES * L)

            def j_body(j, acc):
                off = base + j * (UNROLL * LANES)
                for u in range(UNROLL):
                    acc = acc + vals_v[pl.ds(off + u * LANES, LANES)]
                return acc

            acc = lax.fori_loop(0, L // UNROLL, j_body,
                                jnp.zeros((LANES,), jnp.float32))
            y = acc * (1.0 / L)
            p = 1.0 / (1.0 + jnp.exp(-y))
            scaled = p * 10000.0
            r = ((scaled + 8388608.0) - 8388608.0) / 10000.0
            outs_v[pl.ds(c * CHUNK_ROWS + g * LANES, LANES)] = r
            return 0

        lax.fori_loop(0, CHUNK_ROWS // LANES, group_body, 0)

    bufs = [(idx_a, vals_a, sem_a), (idx_b, vals_b, sem_b)]
    desc = fetch(0, *bufs[0])
    for c in range(N_CHUNKS):
        if c + 1 < N_CHUNKS:
            nxt = fetch(c + 1, *bufs[(c + 1) % 2])
        desc.wait()
        compute(c, bufs[c % 2][1])
        if c + 1 < N_CHUNKS:
            desc = nxt

    pltpu.sync_copy(outs_v, out_hbm.at[pl.ds(row0, ROWS_PER_W)])


def _sc_pool(t, xt):
    mesh = plsc.VectorSubcoreMesh(
        core_axis_name="c", subcore_axis_name="s",
        num_cores=NC, num_subcores=NS)
    run = functools.partial(
        pl.kernel,
        out_type=jax.ShapeDtypeStruct((B,), jnp.float32),
        mesh=mesh,
        scratch_types=[
            pltpu.VMEM((CHUNK_IDX,), jnp.int32),
            pltpu.VMEM((CHUNK_IDX,), jnp.int32),
            pltpu.VMEM((CHUNK_IDX,), jnp.float32),
            pltpu.VMEM((CHUNK_IDX,), jnp.float32),
            pltpu.VMEM((ROWS_PER_W,), jnp.float32),
            pltpu.SemaphoreType.DMA,
            pltpu.SemaphoreType.DMA,
        ],
    )(_sc_body)
    return run(t, xt)


def kernel(x, emb_table, W, b):
    t = _fold_table(emb_table, W, b)
    xt = x.reshape(B // LANES, LANES, L).transpose(0, 2, 1).reshape(B * L)
    out = _sc_pool(t, xt)
    return out.reshape(B, 1)


# speedup vs baseline: 1.2433x; 1.2433x over previous
import functools

import jax
import jax.numpy as jnp
from jax import lax
from jax.experimental import pallas as pl
from jax.experimental.pallas import tpu as pltpu
from jax.experimental.pallas import tpu_sc as plsc

V = 1000000
D = 16
B = 16384
L = 200

NC = 2
NS = 16
LANES = 16
NW = NC * NS
ROWS_PER_W = B // NW
CHUNK_ROWS = 128
N_CHUNKS = ROWS_PER_W // CHUNK_ROWS
CHUNK_IDX = CHUNK_ROWS * L

VP = 1048576
TC_K = 128 * D
TC_ROWS = VP // 128
TC_BLK = 512

UNROLL = 8
assert L % UNROLL == 0


def _table_dot_body(m_ref, b_ref, emb_ref, out_ref):
    out_ref[...] = jnp.dot(emb_ref[...], m_ref[...],
                           preferred_element_type=jnp.float32) + b_ref[0]


def _fold_table(emb_table, W, b):
    m = jnp.kron(jnp.eye(128, dtype=jnp.float32), W)
    flat = jnp.pad(emb_table.reshape(V * D), (0, (VP - V) * D))
    grid = TC_ROWS // TC_BLK
    out = pl.pallas_call(
        _table_dot_body,
        grid=(grid,),
        in_specs=[
            pl.BlockSpec((TC_K, 128), lambda i: (0, 0)),
            pl.BlockSpec(memory_space=pltpu.SMEM),
            pl.BlockSpec((TC_BLK, TC_K), lambda i: (i, 0)),
        ],
        out_specs=pl.BlockSpec((TC_BLK, 128), lambda i: (i, 0)),
        out_shape=jax.ShapeDtypeStruct((TC_ROWS, 128), jnp.float32),
    )(m, b, flat.reshape(TC_ROWS, TC_K))
    return out.reshape(VP)


def _sc_body(t_hbm, xt_hbm, out_hbm,
             idx_a, idx_b, vals_a, vals_b, outs_v, sem_a, sem_b):
    wid = lax.axis_index("s") * NC + lax.axis_index("c")
    row0 = wid * ROWS_PER_W

    def fetch(c, idx_v, vals_v, sem):
        idx0 = (row0 + c * CHUNK_ROWS) * L
        pltpu.sync_copy(xt_hbm.at[pl.ds(idx0, CHUNK_IDX)], idx_v)
        return pltpu.async_copy(t_hbm.at[idx_v], vals_v, sem)

    def compute(c, vals_v):
        def group_body(g, _):
            base = g * (LANES * L)

            def j_body(j, acc):
                off = base + j * (UNROLL * LANES)
                for u in range(UNROLL):
                    acc = acc + vals_v[pl.ds(off + u * LANES, LANES)]
                return acc

            acc = lax.fori_loop(0, L // UNROLL, j_body,
                                jnp.zeros((LANES,), jnp.float32))
            y = acc * (1.0 / L)
            p = 1.0 / (1.0 + jnp.exp(-y))
            scaled = p * 10000.0
            r = ((scaled + 8388608.0) - 8388608.0) / 10000.0
            outs_v[pl.ds(c * CHUNK_ROWS + g * LANES, LANES)] = r
            return 0

        lax.fori_loop(0, CHUNK_ROWS // LANES, group_body, 0)

    bufs = [(idx_a, vals_a, sem_a), (idx_b, vals_b, sem_b)]
    desc = fetch(0, *bufs[0])
    for c in range(N_CHUNKS):
        if c + 1 < N_CHUNKS:
            nxt = fetch(c + 1, *bufs[(c + 1) % 2])
        desc.wait()
        compute(c, bufs[c % 2][1])
        if c + 1 < N_CHUNKS:
            desc = nxt

    pltpu.sync_copy(outs_v, out_hbm.at[pl.ds(row0, ROWS_PER_W)])


def _sc_pool(t, xt):
    mesh = plsc.VectorSubcoreMesh(
        core_axis_name="c", subcore_axis_name="s",
        num_cores=NC, num_subcores=NS)
    run = functools.partial(
        pl.kernel,
        out_type=jax.ShapeDtypeStruct((B,), jnp.float32),
        mesh=mesh,
        scratch_types=[
            pltpu.VMEM((CHUNK_IDX,), jnp.int32),
            pltpu.VMEM((CHUNK_IDX,), jnp.int32),
            pltpu.VMEM((CHUNK_IDX,), jnp.float32),
            pltpu.VMEM((CHUNK_IDX,), jnp.float32),
            pltpu.VMEM((ROWS_PER_W,), jnp.float32),
            pltpu.SemaphoreType.DMA,
            pltpu.SemaphoreType.DMA,
        ],
    )(_sc_body)
    return run(t, xt)


def kernel(x, emb_table, W, b):
    t = _fold_table(emb_table, W, b)
    xt = x.reshape(B // LANES, LANES, L).transpose(0, 2, 1).reshape(B * L)
    out = _sc_pool(t, xt)
    return out.reshape(B, 1)


# speedup vs baseline: 1.3003x; 1.0458x over previous
import functools

import jax
import jax.numpy as jnp
from jax import lax
from jax.experimental import pallas as pl
from jax.experimental.pallas import tpu as pltpu
from jax.experimental.pallas import tpu_sc as plsc

V = 1000000
D = 16
B = 16384
L = 200

NC = 2
NS = 16
LANES = 16
NW = NC * NS
ROWS_PER_W = B // NW
CHUNK_ROWS = 128
N_CHUNKS = ROWS_PER_W // CHUNK_ROWS
CHUNK_IDX = CHUNK_ROWS * L
PAIR_W = 2 * L
N_PAIRS = CHUNK_ROWS // 2

VP = 1048576
TC_K = 128 * D
TC_ROWS = VP // 128
TC_BLK = 512


def _table_dot_body(m_ref, emb_ref, out_ref):
    out_ref[...] = jnp.dot(emb_ref[...], m_ref[...],
                           preferred_element_type=jnp.float32)


def _fold_table(emb_table, W):
    m = jnp.kron(jnp.eye(128, dtype=jnp.float32), W)
    flat = jnp.pad(emb_table.reshape(V * D), (0, (VP - V) * D))
    grid = TC_ROWS // TC_BLK
    out = pl.pallas_call(
        _table_dot_body,
        grid=(grid,),
        in_specs=[
            pl.BlockSpec((TC_K, 128), lambda i: (0, 0)),
            pl.BlockSpec((TC_BLK, TC_K), lambda i: (i, 0)),
        ],
        out_specs=pl.BlockSpec((TC_BLK, 128), lambda i: (i, 0)),
        out_shape=jax.ShapeDtypeStruct((TC_ROWS, 128), jnp.float32),
    )(m, flat.reshape(TC_ROWS, TC_K))
    return out.reshape(VP)


def _sc_body(t_hbm, xf_hbm, out_hbm,
             idx_a, idx_b, vals_a, vals_b, outs_v, sem_a, sem_b):
    wid = lax.axis_index("s") * NC + lax.axis_index("c")
    row0 = wid * ROWS_PER_W
    lane = lax.iota(jnp.int32, LANES)
    lo = lane < 8

    def fetch(c, idx_v, vals_v, sem):
        idx0 = (row0 + c * CHUNK_ROWS) * L
        pltpu.sync_copy(xf_hbm.at[pl.ds(idx0, CHUNK_IDX)], idx_v)
        return pltpu.async_copy(t_hbm.at[idx_v], vals_v, sem)

    def compute(c, vals_v):
        def pair_body(p, _):
            base = p * PAIR_W
            vs = [vals_v[pl.ds(base + k * LANES, LANES)] for k in range(25)]
            acc_e = vs[0]
            for k in range(1, 12):
                acc_e = acc_e + vs[k]
            acc_o = vs[13]
            for k in range(14, 25):
                acc_o = acc_o + vs[k]
            zero = jnp.zeros((LANES,), jnp.float32)
            acc_e = acc_e + jnp.where(lo, vs[12], zero)
            acc_o = acc_o + jnp.where(lo, zero, vs[12])
            o0 = (c * CHUNK_ROWS + 2 * p) * D
            outs_v[pl.ds(o0, D)] = acc_e
            outs_v[pl.ds(o0 + D, D)] = acc_o
            return 0

        lax.fori_loop(0, N_PAIRS, pair_body, 0)

    bufs = [(idx_a, vals_a, sem_a), (idx_b, vals_b, sem_b)]
    desc = fetch(0, *bufs[0])
    for c in range(N_CHUNKS):
        if c + 1 < N_CHUNKS:
            nxt = fetch(c + 1, *bufs[(c + 1) % 2])
        desc.wait()
        compute(c, bufs[c % 2][1])
        if c + 1 < N_CHUNKS:
            desc = nxt

    pltpu.sync_copy(outs_v, out_hbm.at[pl.ds(row0 * D, ROWS_PER_W * D)])


def _sc_pool(t, xf):
    mesh = plsc.VectorSubcoreMesh(
        core_axis_name="c", subcore_axis_name="s",
        num_cores=NC, num_subcores=NS)
    run = functools.partial(
        pl.kernel,
        out_type=jax.ShapeDtypeStruct((B * D,), jnp.float32),
        mesh=mesh,
        scratch_types=[
            pltpu.VMEM((CHUNK_IDX,), jnp.int32),
            pltpu.VMEM((CHUNK_IDX,), jnp.int32),
            pltpu.VMEM((CHUNK_IDX,), jnp.float32),
            pltpu.VMEM((CHUNK_IDX,), jnp.float32),
            pltpu.VMEM((ROWS_PER_W * D,), jnp.float32),
            pltpu.SemaphoreType.DMA,
            pltpu.SemaphoreType.DMA,
        ],
    )(_sc_body)
    return run(t, xf)


FIN_COLS = 8
FIN_ROWS = B // FIN_COLS
FIN_K = FIN_COLS * D


def _finish_body(m_ref, b_ref, s_ref, out_ref):
    y = jnp.dot(s_ref[...], m_ref[...],
                preferred_element_type=jnp.float32) + b_ref[0]
    p = 1.0 / (1.0 + jnp.exp(-y))
    scaled = p * 10000.0
    out_ref[...] = ((scaled + 8388608.0) - 8388608.0) / 10000.0


def _finish(sums, b):
    m = jnp.kron(jnp.eye(FIN_COLS, dtype=jnp.float32),
                 jnp.full((D, 1), 1.0 / L, jnp.float32))
    out = pl.pallas_call(
        _finish_body,
        in_specs=[
            pl.BlockSpec((FIN_K, FIN_COLS), lambda: (0, 0)),
            pl.BlockSpec(memory_space=pltpu.SMEM),
            pl.BlockSpec((FIN_ROWS, FIN_K), lambda: (0, 0)),
        ],
        out_specs=pl.BlockSpec((FIN_ROWS, FIN_COLS), lambda: (0, 0)),
        out_shape=jax.ShapeDtypeStruct((FIN_ROWS, FIN_COLS), jnp.float32),
    )(m, b, sums.reshape(FIN_ROWS, FIN_K))
    return out.reshape(B, 1)


def kernel(x, emb_table, W, b):
    xf = x.reshape(B * L)
    t = _fold_table(emb_table, W)
    sums = _sc_pool(t, xf)
    return _finish(sums, b)


# speedup vs baseline: 1.4775x; 1.1363x over previous
import functools

import jax
import jax.numpy as jnp
from jax import lax
from jax.experimental import pallas as pl
from jax.experimental.pallas import tpu as pltpu
from jax.experimental.pallas import tpu_sc as plsc

V = 1000000
D = 16
B = 16384
L = 200

NC = 2
NS = 16
LANES = 16
NW = NC * NS
ROWS_PER_W = B // NW
CHUNK_ROWS = 64
N_CHUNKS = ROWS_PER_W // CHUNK_ROWS
CHUNK_IDX = CHUNK_ROWS * L
PAIR_W = 2 * L
N_PAIRS = CHUNK_ROWS // 2

VP = 1048576
TC_K = 128 * D
TC_ROWS = VP // 128
TC_BLK = 512


def _table_dot_body(m_ref, emb_ref, out_ref):
    out_ref[...] = jnp.dot(emb_ref[...], m_ref[...],
                           preferred_element_type=jnp.float32)


def _fold_table(emb_table, W):
    m = jnp.kron(jnp.eye(128, dtype=jnp.float32), W)
    flat = jnp.pad(emb_table.reshape(V * D), (0, (VP - V) * D))
    grid = TC_ROWS // TC_BLK
    out = pl.pallas_call(
        _table_dot_body,
        grid=(grid,),
        in_specs=[
            pl.BlockSpec((TC_K, 128), lambda i: (0, 0)),
            pl.BlockSpec((TC_BLK, TC_K), lambda i: (i, 0)),
        ],
        out_specs=pl.BlockSpec((TC_BLK, 128), lambda i: (i, 0)),
        out_shape=jax.ShapeDtypeStruct((TC_ROWS, 128), jnp.float32),
    )(m, flat.reshape(TC_ROWS, TC_K))
    return out.reshape(VP)


def _sc_body(t_hbm, xf_hbm, out_hbm,
             idx_a, idx_b, vals_a, vals_b, outs_v, t_sh, sem_a, sem_b):
    wid = lax.axis_index("s") * NC + lax.axis_index("c")
    row0 = wid * ROWS_PER_W
    lane = lax.iota(jnp.int32, LANES)
    lo = lane < 8

    sid = lax.axis_index("s")
    shard = VP // NS
    pltpu.sync_copy(t_hbm.at[pl.ds(sid * shard, shard)],
                    t_sh.at[pl.ds(sid * shard, shard)])
    plsc.subcore_barrier()

    def fetch(c, idx_v, vals_v, sem):
        idx0 = (row0 + c * CHUNK_ROWS) * L
        pltpu.sync_copy(xf_hbm.at[pl.ds(idx0, CHUNK_IDX)], idx_v)
        return pltpu.async_copy(t_sh.at[idx_v], vals_v, sem)

    def compute(c, vals_v):
        def pair_body(p, _):
            base = p * PAIR_W
            vs = [vals_v[pl.ds(base + k * LANES, LANES)] for k in range(25)]
            acc_e = vs[0]
            for k in range(1, 12):
                acc_e = acc_e + vs[k]
            acc_o = vs[13]
            for k in range(14, 25):
                acc_o = acc_o + vs[k]
            zero = jnp.zeros((LANES,), jnp.float32)
            acc_e = acc_e + jnp.where(lo, vs[12], zero)
            acc_o = acc_o + jnp.where(lo, zero, vs[12])
            o0 = (c * CHUNK_ROWS + 2 * p) * D
            outs_v[pl.ds(o0, D)] = acc_e
            outs_v[pl.ds(o0 + D, D)] = acc_o
            return 0

        lax.fori_loop(0, N_PAIRS, pair_body, 0)

    bufs = [(idx_a, vals_a, sem_a), (idx_b, vals_b, sem_b)]
    desc = fetch(0, *bufs[0])
    for c in range(N_CHUNKS):
        if c + 1 < N_CHUNKS:
            nxt = fetch(c + 1, *bufs[(c + 1) % 2])
        desc.wait()
        compute(c, bufs[c % 2][1])
        if c + 1 < N_CHUNKS:
            desc = nxt

    pltpu.sync_copy(outs_v, out_hbm.at[pl.ds(row0 * D, ROWS_PER_W * D)])


def _sc_pool(t, xf):
    mesh = plsc.VectorSubcoreMesh(
        core_axis_name="c", subcore_axis_name="s",
        num_cores=NC, num_subcores=NS)
    run = functools.partial(
        pl.kernel,
        out_type=jax.ShapeDtypeStruct((B * D,), jnp.float32),
        mesh=mesh,
        scratch_types=[
            pltpu.VMEM((CHUNK_IDX,), jnp.int32),
            pltpu.VMEM((CHUNK_IDX,), jnp.int32),
            pltpu.VMEM((CHUNK_IDX,), jnp.float32),
            pltpu.VMEM((CHUNK_IDX,), jnp.float32),
            pltpu.VMEM((ROWS_PER_W * D,), jnp.float32),
            pltpu.VMEM_SHARED((VP,), jnp.float32),
            pltpu.SemaphoreType.DMA,
            pltpu.SemaphoreType.DMA,
        ],
    )(_sc_body)
    return run(t, xf)


FIN_COLS = 8
FIN_ROWS = B // FIN_COLS
FIN_K = FIN_COLS * D


def _finish_body(m_ref, b_ref, s_ref, out_ref):
    y = jnp.dot(s_ref[...], m_ref[...],
                preferred_element_type=jnp.float32) + b_ref[0]
    p = 1.0 / (1.0 + jnp.exp(-y))
    scaled = p * 10000.0
    out_ref[...] = ((scaled + 8388608.0) - 8388608.0) / 10000.0


def _finish(sums, b):
    m = jnp.kron(jnp.eye(FIN_COLS, dtype=jnp.float32),
                 jnp.full((D, 1), 1.0 / L, jnp.float32))
    out = pl.pallas_call(
        _finish_body,
        in_specs=[
            pl.BlockSpec((FIN_K, FIN_COLS), lambda: (0, 0)),
            pl.BlockSpec(memory_space=pltpu.SMEM),
            pl.BlockSpec((FIN_ROWS, FIN_K), lambda: (0, 0)),
        ],
        out_specs=pl.BlockSpec((FIN_ROWS, FIN_COLS), lambda: (0, 0)),
        out_shape=jax.ShapeDtypeStruct((FIN_ROWS, FIN_COLS), jnp.float32),
    )(m, b, sums.reshape(FIN_ROWS, FIN_K))
    return out.reshape(B, 1)


def kernel(x, emb_table, W, b):
    xf = x.reshape(B * L)
    t = _fold_table(emb_table, W)
    sums = _sc_pool(t, xf)
    return _finish(sums, b)
